# baseline (speedup 1.0000x reference)
import functools

import jax
import jax.numpy as jnp
from jax import lax
from jax.experimental import pallas as pl
from jax.experimental.pallas import tpu as pltpu
from jax.experimental.pallas import tpu_sc as plsc

N = 10000
HID = 128
POS_DIM = 100

_NODE_BLK = 1000
_EDGE_BLK = 2048

_NC = 2
_NS = 16
_NW = _NC * _NS
_E2 = 344064
_CH = 96
_BPW = _E2 // _NW
_NCHUNK = _BPW // _CH
_NCH0 = 112
_NCH1 = 112
_IDXROWS = 3584

_PACK_PERM = tuple(
    32 * c + 16 * (i % 2) + i // 2 for c in range(4) for i in range(32))


def _gather_z(a, b, src2d, dst2d):
    mesh = plsc.VectorSubcoreMesh(core_axis_name="c", subcore_axis_name="s")

    @functools.partial(
        pl.kernel,
        out_type=jax.ShapeDtypeStruct((_E2, HID), jnp.float32),
        mesh=mesh,
        scratch_types=[
            pltpu.VMEM((_NCH0, _CH), jnp.int32),
            pltpu.VMEM((_NCH0, _CH), jnp.int32),
            pltpu.VMEM((_CH, HID), jnp.float32),
            pltpu.VMEM((_CH, HID), jnp.float32),
            pltpu.VMEM((_CH, HID), jnp.float32),
            pltpu.VMEM((_CH, HID), jnp.float32),
            pltpu.SemaphoreType.DMA,
            pltpu.SemaphoreType.DMA,
        ],
    )
    def k(a_hbm, b_hbm, src_hbm, dst_hbm, z_hbm, sidx, didx, arows0, brows0,
          arows1, brows1, sg0, sg1):
        c = lax.axis_index("c")
        s = lax.axis_index("s")
        nchunk = jnp.where(c == 0, _NCH0, _NCH1)
        rowbase = jnp.where(c == 0, s * _NCH0, 16 * _NCH0 + s * _NCH1)
        ebase = rowbase * _CH
        pltpu.sync_copy(src_hbm.at[pl.ds(rowbase, _NCH0)], sidx)
        pltpu.sync_copy(dst_hbm.at[pl.ds(rowbase, _NCH0)], didx)

        nhalf = nchunk // 2

        def start(i, ar, br, sem):
            pltpu.async_copy(a_hbm.at[sidx.at[i]], ar, sem)
            pltpu.async_copy(b_hbm.at[didx.at[i]], br, sem)

        def drain(ar, br, sem):
            pltpu.make_async_copy(a_hbm.at[pl.ds(0, _CH)], ar, sem).wait()
            pltpu.make_async_copy(b_hbm.at[pl.ds(0, _CH)], br, sem).wait()

        def proc(i, ar, br):
            def row4(r, c2):
                r4 = r * 4
                for rr in range(4):
                    for cc in range(HID // 16):
                        av = ar[r4 + rr, pl.ds(cc * 16, 16)]
                        bv = br[r4 + rr, pl.ds(cc * 16, 16)]
                        ar[r4 + rr, pl.ds(cc * 16, 16)] = jnp.maximum(
                            av - bv, 0.0)
                return c2

            lax.fori_loop(0, _CH // 4, row4, 0)
            pltpu.sync_copy(ar, z_hbm.at[pl.ds(ebase + i * _CH, _CH)])

        start(0, arows0, brows0, sg0)

        def iter2(j, carry):
            start(2 * j + 1, arows1, brows1, sg1)
            drain(arows0, brows0, sg0)
            proc(2 * j, arows0, brows0)

            @pl.when(j + 1 < nhalf)
            def _():
                start(2 * j + 2, arows0, brows0, sg0)

            drain(arows1, brows1, sg1)
            proc(2 * j + 1, arows1, brows1)
            return carry

        lax.fori_loop(0, nhalf, iter2, 0)

    return k(a, b, src2d, dst2d)


def _ab_kernel(h_ref, pos_ref, wx_ref, wp_ref, b1_ref, a_ref, b_ref):
    pos_proj = jax.lax.dot_general(
        pos_ref[...], wp_ref[...], (((1,), (1,)), ((), ())),
        preferred_element_type=jnp.float32)
    b_ref[...] = pos_proj
    a_ref[...] = jax.lax.dot_general(
        h_ref[...], wx_ref[...], (((1,), (1,)), ((), ())),
        preferred_element_type=jnp.float32) + pos_proj + b1_ref[...]


def _node_precompute(h, pos, lw1, lb1):
    ind = h.shape[1]
    wx = lw1[:, :ind]
    wp = lw1[:, ind:]
    grid = N // _NODE_BLK
    return pl.pallas_call(
        _ab_kernel,
        grid=(grid,),
        in_specs=[
            pl.BlockSpec((_NODE_BLK, ind), lambda i: (i, 0)),
            pl.BlockSpec((_NODE_BLK, POS_DIM), lambda i: (i, 0)),
            pl.BlockSpec((HID, ind), lambda i: (0, 0)),
            pl.BlockSpec((HID, POS_DIM), lambda i: (0, 0)),
            pl.BlockSpec((1, HID), lambda i: (0, 0)),
        ],
        out_specs=[
            pl.BlockSpec((_NODE_BLK, HID), lambda i: (i, 0)),
            pl.BlockSpec((_NODE_BLK, HID), lambda i: (i, 0)),
        ],
        out_shape=[
            jax.ShapeDtypeStruct((N, HID), jnp.float32),
            jax.ShapeDtypeStruct((N, HID), jnp.float32),
        ],
    )(h, pos, wx, wp, lb1.reshape(1, HID))


def _edge_mm_kernel(z_ref, w2_ref, m_ref):
    mt = jax.lax.dot_general(
        w2_ref[...], z_ref[...], (((1,), (1,)), ((), ())),
        preferred_element_type=jnp.float32)
    lo = lax.bitcast_convert_type(
        mt[: HID // 2].astype(jnp.bfloat16), jnp.uint16).astype(jnp.uint32)
    hi = lax.bitcast_convert_type(
        mt[HID // 2:].astype(jnp.bfloat16), jnp.uint16).astype(jnp.uint32)
    m_ref[...] = lax.bitcast_convert_type((hi << 16) | lo, jnp.int32)


def _edge_matmul_packed(z, lw2):
    e = z.shape[0]
    grid = e // _EDGE_BLK
    return pl.pallas_call(
        _edge_mm_kernel,
        grid=(grid,),
        in_specs=[
            pl.BlockSpec((_EDGE_BLK, HID), lambda i: (i, 0)),
            pl.BlockSpec((HID, HID), lambda i: (0, 0)),
        ],
        out_specs=pl.BlockSpec((HID // 2, _EDGE_BLK), lambda i: (0, i)),
        out_shape=jax.ShapeDtypeStruct((HID // 2, e), jnp.int32),
    )(z, lw2)


_NPAD = 10240
_CHE = 1536
_EHALF = _E2 // 2
_NSCCH = _EHALF // _CHE
_CPT = 8


_WPT = 4
_NEGPAIR = -8323200


def _scatter_max(m_p_flat, dst_s):
    mesh = plsc.VectorSubcoreMesh(core_axis_name="c", subcore_axis_name="s")

    @functools.partial(
        pl.kernel,
        out_type=jax.ShapeDtypeStruct((HID * _NPAD,), jnp.float32),
        mesh=mesh,
        compiler_params=pltpu.CompilerParams(needs_layout_passes=False),
        scratch_types=(
            [pltpu.VMEM((1, _NPAD), jnp.int32)] * _WPT
            + [
                pltpu.VMEM((_WPT, _CHE), jnp.int32),
                pltpu.VMEM((_WPT, _CHE), jnp.int32),
                pltpu.VMEM((_CHE,), jnp.int32),
                pltpu.VMEM((_CHE,), jnp.int32),
                pltpu.VMEM((_WPT, 1024), jnp.int32),
                pltpu.VMEM((2, 1024), jnp.float32),
                pltpu.VMEM_SHARED((8, _WPT, 1024), jnp.int32),
                pltpu.SemaphoreType.DMA,
                pltpu.SemaphoreType.DMA,
            ]
        ),
    )
    def k(mp_hbm, dst_hbm, out_hbm, a0, a1, a2, a3,
          slab0, slab1, idxb0, idxb1, mbuf, ubuf, shared, sem0, sem1):
        accs = [a0, a1, a2, a3]
        c = lax.axis_index("c")
        s = lax.axis_index("s")
        g = s // 2
        h = s % 2
        wbase = c * 32 + g * _WPT
        lanes_f = lax.iota(jnp.int32, 16).astype(jnp.float32)
        negw = jnp.full((16,), _NEGPAIR, jnp.int32)
        zz = jnp.zeros((16,), jnp.int32)
        himask = jnp.full((16,), -65536, jnp.int32)
        ebase = h * _EHALF

        def pmax(wa, wb):
            m = jnp.maximum(plsc.bitcast(wa, jnp.bfloat16),
                            plsc.bitcast(wb, jnp.bfloat16))
            return plsc.bitcast(m, jnp.int32)

        def ini(j, carry):
            for w in range(_WPT):
                accs[w][0, pl.ds(j * 16, 16)] = negw
            return carry
        lax.fori_loop(0, _NPAD // 16, ini, 0)

        _GB = 4

        def make_grp_body(idxb, slab):
            def grp_body(q, carry):
                base = q * (16 * _GB)
                idxs = []
                nodup = None
                for t in range(_GB):
                    idx = idxb[pl.ds(base + t * 16, 16)]
                    idxs.append(idx)
                    _, lastm = plsc.scan_count(idx)
                    nd = jnp.all(lastm)
                    nodup = nd if nodup is None else (nodup & nd)
                for t in range(_GB):
                    for w in range(_WPT):
                        vals = slab[w, pl.ds(base + t * 16, 16)]
                        cur = plsc.load_gather(accs[w], [zz, idxs[t]])
                        new = pmax(vals, cur)
                        plsc.store_scatter(accs[w], [zz, idxs[t]], new,
                                           mask=new != cur)

                @pl.when(jnp.logical_not(nodup))
                def _():
                    for t in range(_GB):
                        for w in range(_WPT):
                            vals = slab[w, pl.ds(base + t * 16, 16)]
                            cur = plsc.load_gather(accs[w], [zz, idxs[t]])

                            def body(p, vals=vals, w=w, t=t):
                                c1 = plsc.load_gather(accs[w], [zz, idxs[t]])
                                plsc.store_scatter(accs[w], [zz, idxs[t]],
                                                   pmax(vals, c1), mask=p)
                                c2 = plsc.load_gather(accs[w], [zz, idxs[t]])
                                return pmax(vals, c2) != c2

                            lax.while_loop(jnp.any, body,
                                           pmax(vals, cur) != cur)

                return carry
            return grp_body

        def start(ci, ib, sb, sem):
            eoff = ebase + ci * _CHE
            pltpu.async_copy(dst_hbm.at[pl.ds(eoff, _CHE)], ib, sem)
            for w in range(_WPT):
                pltpu.async_copy(
                    mp_hbm.at[pl.ds((wbase + w) * _E2 + eoff, _CHE)],
                    sb.at[w], sem)

        def drain(ib, sb, sem):
            pltpu.make_async_copy(
                dst_hbm.at[pl.ds(0, _CHE)], ib, sem).wait()
            for w in range(_WPT):
                pltpu.make_async_copy(
                    dst_hbm.at[pl.ds(0, _CHE)], sb.at[w], sem).wait()

        proc0 = make_grp_body(idxb0, slab0)
        proc1 = make_grp_body(idxb1, slab1)

        start(0, idxb0, slab0, sem0)
        drain(idxb0, slab0, sem0)

        def iter2(j, carry):
            start(2 * j + 1, idxb1, slab1, sem1)
            lax.fori_loop(0, _CHE // (16 * _GB), proc0, 0)
            drain(idxb1, slab1, sem1)

            @pl.when(j + 1 < _NSCCH // 2)
            def _():
                start(2 * j + 2, idxb0, slab0, sem0)

            lax.fori_loop(0, _CHE // (16 * _GB), proc1, 0)

            @pl.when(j + 1 < _NSCCH // 2)
            def _():
                drain(idxb0, slab0, sem0)

            return carry

        lax.fori_loop(0, _NSCCH // 2, iter2, 0)

        for j in range(_NPAD // 1024):
            @pl.when(h == 1)
            def _(j=j):
                for w in range(_WPT):
                    pltpu.sync_copy(
                        accs[w].at[0, pl.ds(j * 1024, 1024)],
                        shared.at[g, w])

            plsc.subcore_barrier()

            @pl.when(h == 0)
            def _(j=j):
                pltpu.sync_copy(shared.at[g], mbuf)
                def mx(r, carry2, j=j):
                    for w in range(_WPT):
                        v = mbuf[w, pl.ds(r * 16, 16)]
                        a = accs[w][0, pl.ds(j * 1024 + r * 16, 16)]
                        accs[w][0, pl.ds(j * 1024 + r * 16, 16)] = (
                            pmax(a, v))
                    return carry2
                lax.fori_loop(0, 1024 // 16, mx, 0)

            plsc.subcore_barrier()

        @pl.when(h == 0)
        def _():
            for w in range(_WPT):
                for j in range(_NPAD // 1024):
                    def unp(r, carry2, w=w, j=j):
                        word = accs[w][0, pl.ds(j * 1024 + r * 16, 16)]
                        ubuf[0, pl.ds(r * 16, 16)] = plsc.bitcast(
                            word << 16, jnp.float32)
                        ubuf[1, pl.ds(r * 16, 16)] = plsc.bitcast(
                            word & himask, jnp.float32)
                        return carry2
                    lax.fori_loop(0, 1024 // 16, unp, 0)
                    pltpu.sync_copy(
                        ubuf.at[0],
                        out_hbm.at[pl.ds((wbase + w) * _NPAD + j * 1024,
                                         1024)])
                    pltpu.sync_copy(
                        ubuf.at[1],
                        out_hbm.at[pl.ds((wbase + w + 64) * _NPAD + j * 1024,
                                         1024)])

    return k(m_p_flat, dst_s).reshape(HID, _NPAD)


def _global_mlp_kernel(aggr_ref, b2_ref, gw1_ref, gb1_ref, gw2_ref, gb2_ref,
                       o_ref):
    a = aggr_ref[...] + b2_ref[...]
    t = jnp.maximum(
        jax.lax.dot_general(gw1_ref[...], a, (((1,), (0,)), ((), ())),
                            preferred_element_type=jnp.float32)
        + gb1_ref[...], 0.0)
    o_ref[...] = jax.lax.dot_general(
        t, gw2_ref[...], (((0,), (1,)), ((), ())),
        preferred_element_type=jnp.float32) + gb2_ref[...]


def _global_mlp_t(aggr_t, lb2, gw1, gb1, gw2, gb2):
    blk = 1024
    grid = _NPAD // blk
    return pl.pallas_call(
        _global_mlp_kernel,
        grid=(grid,),
        in_specs=[
            pl.BlockSpec((HID, blk), lambda i: (0, i)),
            pl.BlockSpec((HID, 1), lambda i: (0, 0)),
            pl.BlockSpec((HID, HID), lambda i: (0, 0)),
            pl.BlockSpec((HID, 1), lambda i: (0, 0)),
            pl.BlockSpec((HID, HID), lambda i: (0, 0)),
            pl.BlockSpec((1, HID), lambda i: (0, 0)),
        ],
        out_specs=pl.BlockSpec((blk, HID), lambda i: (i, 0)),
        out_shape=jax.ShapeDtypeStruct((_NPAD, HID), jnp.float32),
    )(aggr_t, lb2.reshape(HID, 1), gw1, gb1.reshape(HID, 1), gw2,
      gb2.reshape(1, HID))


def _layer(h, pos, src2d, dstg2d, dst_s, lw1, lb1, lw2, lb2, gw1, gb1, gw2,
           gb2):
    a, b = _node_precompute(h, pos, lw1, lb1)
    z = _gather_z(a, b, src2d, dstg2d)
    m_p = _edge_matmul_packed(z, lw2)
    aggr_t = _scatter_max(m_p.reshape(-1), dst_s)
    return _global_mlp_t(aggr_t, lb2, gw1, gb1, gw2, gb2)[:N]


def kernel(x, pos, edge_index, l0_lw1, l0_lb1, l0_lw2, l0_lb2, l0_gw1,
           l0_gb1, l0_gw2, l0_gb2, l1_lw1, l1_lb1, l1_lw2, l1_lb2, l1_gw1,
           l1_gb1, l1_gw2, l1_gb2):
    n = x.shape[0]
    e = edge_index.shape[1]
    npad = _E2 - e - n
    nidx = _IDXROWS * _CH - e - n
    loop = jnp.arange(n, dtype=edge_index.dtype)
    src = jnp.concatenate(
        [edge_index[0], loop, jnp.zeros((nidx,), edge_index.dtype)])
    dst = jnp.concatenate([edge_index[1], loop])
    dst_g = jnp.concatenate([dst, jnp.zeros((nidx,), edge_index.dtype)])
    dst_s = jnp.concatenate([dst, jnp.full((npad,), n, edge_index.dtype)])
    src2d = src.reshape(_IDXROWS, _CH)
    dstg2d = dst_g.reshape(_IDXROWS, _CH)
    h = _layer(x, pos, src2d, dstg2d, dst_s, l0_lw1, l0_lb1, l0_lw2, l0_lb2,
               l0_gw1, l0_gb1, l0_gw2, l0_gb2)
    h = _layer(h, pos, src2d, dstg2d, dst_s, l1_lw1, l1_lb1, l1_lw2, l1_lb2,
               l1_gw1, l1_gb1, l1_gw2, l1_gb2)
    return h


# speedup vs baseline: 1.2929x; 1.2929x over previous
import functools

import jax
import jax.numpy as jnp
from jax import lax
from jax.experimental import pallas as pl
from jax.experimental.pallas import tpu as pltpu
from jax.experimental.pallas import tpu_sc as plsc

N = 10000
HID = 128
POS_DIM = 100

_NODE_BLK = 1000
_EDGE_BLK = 2048

_NC = 2
_NS = 16
_NW = _NC * _NS
_E2 = 337920
_CH = 120
_BPW = _E2 // _NW
_NCHUNK = _BPW // _CH
_NCH0 = 88
_NCH1 = 88
_IDXROWS = 2816


def _gather_z(a, b, src2d, dst2d):
    mesh = plsc.VectorSubcoreMesh(core_axis_name="c", subcore_axis_name="s")

    @functools.partial(
        pl.kernel,
        out_type=jax.ShapeDtypeStruct((_E2, HID), jnp.float32),
        mesh=mesh,
        scratch_types=[
            pltpu.VMEM((_NCH0, _CH), jnp.int32),
            pltpu.VMEM((_NCH0, _CH), jnp.int32),
            pltpu.VMEM((_CH, HID), jnp.float32),
            pltpu.VMEM((_CH, HID), jnp.float32),
            pltpu.VMEM((_CH, HID), jnp.float32),
            pltpu.VMEM((_CH, HID), jnp.float32),
            pltpu.SemaphoreType.DMA,
            pltpu.SemaphoreType.DMA,
        ],
    )
    def k(a_hbm, b_hbm, src_hbm, dst_hbm, z_hbm, sidx, didx, arows0, brows0,
          arows1, brows1, sg0, sg1):
        c = lax.axis_index("c")
        s = lax.axis_index("s")
        nchunk = jnp.where(c == 0, _NCH0, _NCH1)
        rowbase = jnp.where(c == 0, s * _NCH0, 16 * _NCH0 + s * _NCH1)
        ebase = rowbase * _CH
        pltpu.sync_copy(src_hbm.at[pl.ds(rowbase, _NCH0)], sidx)
        pltpu.sync_copy(dst_hbm.at[pl.ds(rowbase, _NCH0)], didx)

        nhalf = nchunk // 2

        def start(i, ar, br, sem):
            pltpu.async_copy(a_hbm.at[sidx.at[i]], ar, sem)
            pltpu.async_copy(b_hbm.at[didx.at[i]], br, sem)

        def drain(ar, br, sem):
            pltpu.make_async_copy(a_hbm.at[pl.ds(0, _CH)], ar, sem).wait()
            pltpu.make_async_copy(b_hbm.at[pl.ds(0, _CH)], br, sem).wait()

        def proc(i, ar, br):
            def row4(r, c2):
                r4 = r * 4
                for rr in range(4):
                    for cc in range(HID // 16):
                        av = ar[r4 + rr, pl.ds(cc * 16, 16)]
                        bv = br[r4 + rr, pl.ds(cc * 16, 16)]
                        ar[r4 + rr, pl.ds(cc * 16, 16)] = jnp.maximum(
                            av - bv, 0.0)
                return c2

            lax.fori_loop(0, _CH // 4, row4, 0)
            pltpu.sync_copy(ar, z_hbm.at[pl.ds(ebase + i * _CH, _CH)])

        start(0, arows0, brows0, sg0)

        def iter2(j, carry):
            start(2 * j + 1, arows1, brows1, sg1)
            drain(arows0, brows0, sg0)
            proc(2 * j, arows0, brows0)

            @pl.when(j + 1 < nhalf)
            def _():
                start(2 * j + 2, arows0, brows0, sg0)

            drain(arows1, brows1, sg1)
            proc(2 * j + 1, arows1, brows1)
            return carry

        lax.fori_loop(0, nhalf, iter2, 0)

    return k(a, b, src2d, dst2d)


def _ab_kernel(h_ref, pos_ref, wx_ref, wp_ref, b1_ref, a_ref, b_ref):
    pos_proj = jax.lax.dot_general(
        pos_ref[...], wp_ref[...], (((1,), (1,)), ((), ())),
        preferred_element_type=jnp.float32)
    b_ref[...] = pos_proj
    a_ref[...] = jax.lax.dot_general(
        h_ref[...], wx_ref[...], (((1,), (1,)), ((), ())),
        preferred_element_type=jnp.float32) + pos_proj + b1_ref[...]


def _node_precompute(h, pos, lw1, lb1):
    ind = h.shape[1]
    wx = lw1[:, :ind]
    wp = lw1[:, ind:]
    grid = N // _NODE_BLK
    return pl.pallas_call(
        _ab_kernel,
        grid=(grid,),
        in_specs=[
            pl.BlockSpec((_NODE_BLK, ind), lambda i: (i, 0)),
            pl.BlockSpec((_NODE_BLK, POS_DIM), lambda i: (i, 0)),
            pl.BlockSpec((HID, ind), lambda i: (0, 0)),
            pl.BlockSpec((HID, POS_DIM), lambda i: (0, 0)),
            pl.BlockSpec((1, HID), lambda i: (0, 0)),
        ],
        out_specs=[
            pl.BlockSpec((_NODE_BLK, HID), lambda i: (i, 0)),
            pl.BlockSpec((_NODE_BLK, HID), lambda i: (i, 0)),
        ],
        out_shape=[
            jax.ShapeDtypeStruct((N, HID), jnp.float32),
            jax.ShapeDtypeStruct((N, HID), jnp.float32),
        ],
    )(h, pos, wx, wp, lb1.reshape(1, HID))


def _edge_mm_kernel(z_ref, w2_ref, m_ref):
    mt = jax.lax.dot_general(
        w2_ref[...], z_ref[...], (((1,), (1,)), ((), ())),
        preferred_element_type=jnp.float32)
    lo = lax.bitcast_convert_type(
        mt[: HID // 2].astype(jnp.bfloat16), jnp.uint16).astype(jnp.uint32)
    hi = lax.bitcast_convert_type(
        mt[HID // 2:].astype(jnp.bfloat16), jnp.uint16).astype(jnp.uint32)
    m_ref[...] = lax.bitcast_convert_type((hi << 16) | lo, jnp.int32)


def _edge_matmul_packed(z, lw2):
    e = z.shape[0]
    grid = e // _EDGE_BLK
    return pl.pallas_call(
        _edge_mm_kernel,
        grid=(grid,),
        in_specs=[
            pl.BlockSpec((_EDGE_BLK, HID), lambda i: (i, 0)),
            pl.BlockSpec((HID, HID), lambda i: (0, 0)),
        ],
        out_specs=pl.BlockSpec((HID // 2, _EDGE_BLK), lambda i: (0, i)),
        out_shape=jax.ShapeDtypeStruct((HID // 2, e), jnp.int32),
    )(z, lw2)


_NPAD = 10240
_CHE = 1280
_EHALF = _E2 // 2
_NSCCH = _EHALF // _CHE
_CPT = 8


_WPT = 4
_NEGPAIR = -8323200


def _scatter_max(m_p_flat, dst_s):
    mesh = plsc.VectorSubcoreMesh(core_axis_name="c", subcore_axis_name="s")

    @functools.partial(
        pl.kernel,
        out_type=jax.ShapeDtypeStruct((HID * _NPAD,), jnp.float32),
        mesh=mesh,
        compiler_params=pltpu.CompilerParams(needs_layout_passes=False),
        scratch_types=(
            [pltpu.VMEM((1, _NPAD), jnp.int32)] * _WPT
            + [
                pltpu.VMEM((_WPT, _CHE), jnp.int32),
                pltpu.VMEM((_WPT, _CHE), jnp.int32),
                pltpu.VMEM((_CHE,), jnp.int32),
                pltpu.VMEM((_CHE,), jnp.int32),
                pltpu.VMEM((_WPT, 1024), jnp.int32),
                pltpu.VMEM((2, 1024), jnp.float32),
                pltpu.VMEM_SHARED((8, _WPT, 1024), jnp.int32),
                pltpu.SemaphoreType.DMA,
                pltpu.SemaphoreType.DMA,
            ]
        ),
    )
    def k(mp_hbm, dst_hbm, out_hbm, a0, a1, a2, a3,
          slab0, slab1, idxb0, idxb1, mbuf, ubuf, shared, sem0, sem1):
        accs = [a0, a1, a2, a3]
        c = lax.axis_index("c")
        s = lax.axis_index("s")
        g = s // 2
        h = s % 2
        wbase = c * 32 + g * _WPT
        lanes_f = lax.iota(jnp.int32, 16).astype(jnp.float32)
        negw = jnp.full((16,), _NEGPAIR, jnp.int32)
        zz = jnp.zeros((16,), jnp.int32)
        himask = jnp.full((16,), -65536, jnp.int32)
        ebase = h * _EHALF

        def pmax(wa, wb):
            m = jnp.maximum(plsc.bitcast(wa, jnp.bfloat16),
                            plsc.bitcast(wb, jnp.bfloat16))
            return plsc.bitcast(m, jnp.int32)

        def ini(j, carry):
            for w in range(_WPT):
                accs[w][0, pl.ds(j * 16, 16)] = negw
            return carry
        lax.fori_loop(0, _NPAD // 16, ini, 0)

        _GB = 4

        def make_grp_body(idxb, slab):
            def grp_body(q, carry):
                base = q * (16 * _GB)
                idxs = []
                nodup = None
                for t in range(_GB):
                    idx = idxb[pl.ds(base + t * 16, 16)]
                    idxs.append(idx)
                    _, lastm = plsc.scan_count(idx)
                    nd = jnp.all(lastm)
                    nodup = nd if nodup is None else (nodup & nd)
                for t in range(_GB):
                    for w in range(_WPT):
                        vals = slab[w, pl.ds(base + t * 16, 16)]
                        cur = plsc.load_gather(accs[w], [zz, idxs[t]])
                        new = pmax(vals, cur)
                        plsc.store_scatter(accs[w], [zz, idxs[t]], new,
                                           mask=new != cur)

                @pl.when(jnp.logical_not(nodup))
                def _():
                    for t in range(_GB):
                        for w in range(_WPT):
                            vals = slab[w, pl.ds(base + t * 16, 16)]
                            cur = plsc.load_gather(accs[w], [zz, idxs[t]])

                            def body(p, vals=vals, w=w, t=t):
                                c1 = plsc.load_gather(accs[w], [zz, idxs[t]])
                                plsc.store_scatter(accs[w], [zz, idxs[t]],
                                                   pmax(vals, c1), mask=p)
                                c2 = plsc.load_gather(accs[w], [zz, idxs[t]])
                                return pmax(vals, c2) != c2

                            lax.while_loop(jnp.any, body,
                                           pmax(vals, cur) != cur)

                return carry
            return grp_body

        def start(ci, ib, sb, sem):
            eoff = ebase + ci * _CHE
            pltpu.async_copy(dst_hbm.at[pl.ds(eoff, _CHE)], ib, sem)
            for w in range(_WPT):
                pltpu.async_copy(
                    mp_hbm.at[pl.ds((wbase + w) * _E2 + eoff, _CHE)],
                    sb.at[w], sem)

        def drain(ib, sb, sem):
            pltpu.make_async_copy(
                dst_hbm.at[pl.ds(0, _CHE)], ib, sem).wait()
            for w in range(_WPT):
                pltpu.make_async_copy(
                    dst_hbm.at[pl.ds(0, _CHE)], sb.at[w], sem).wait()

        proc0 = make_grp_body(idxb0, slab0)
        proc1 = make_grp_body(idxb1, slab1)

        start(0, idxb0, slab0, sem0)
        drain(idxb0, slab0, sem0)

        def iter2(j, carry):
            start(2 * j + 1, idxb1, slab1, sem1)
            lax.fori_loop(0, _CHE // (16 * _GB), proc0, 0)
            drain(idxb1, slab1, sem1)

            @pl.when(j + 1 < _NSCCH // 2)
            def _():
                start(2 * j + 2, idxb0, slab0, sem0)

            lax.fori_loop(0, _CHE // (16 * _GB), proc1, 0)

            @pl.when(j + 1 < _NSCCH // 2)
            def _():
                drain(idxb0, slab0, sem0)

            return carry

        lax.fori_loop(0, _NSCCH // 2, iter2, 0)

        for j in range(_NPAD // 1024):
            @pl.when(h == 1)
            def _(j=j):
                for w in range(_WPT):
                    pltpu.sync_copy(
                        accs[w].at[0, pl.ds(j * 1024, 1024)],
                        shared.at[g, w])

            plsc.subcore_barrier()

            @pl.when(h == 0)
            def _(j=j):
                pltpu.sync_copy(shared.at[g], mbuf)
                def mx(r, carry2, j=j):
                    for w in range(_WPT):
                        v = mbuf[w, pl.ds(r * 16, 16)]
                        a = accs[w][0, pl.ds(j * 1024 + r * 16, 16)]
                        accs[w][0, pl.ds(j * 1024 + r * 16, 16)] = (
                            pmax(a, v))
                    return carry2
                lax.fori_loop(0, 1024 // 16, mx, 0)

            plsc.subcore_barrier()

        @pl.when(h == 0)
        def _():
            for w in range(_WPT):
                for j in range(_NPAD // 1024):
                    def unp(r, carry2, w=w, j=j):
                        word = accs[w][0, pl.ds(j * 1024 + r * 16, 16)]
                        ubuf[0, pl.ds(r * 16, 16)] = plsc.bitcast(
                            word << 16, jnp.float32)
                        ubuf[1, pl.ds(r * 16, 16)] = plsc.bitcast(
                            word & himask, jnp.float32)
                        return carry2
                    lax.fori_loop(0, 1024 // 16, unp, 0)
                    pltpu.sync_copy(
                        ubuf.at[0],
                        out_hbm.at[pl.ds((wbase + w) * _NPAD + j * 1024,
                                         1024)])
                    pltpu.sync_copy(
                        ubuf.at[1],
                        out_hbm.at[pl.ds((wbase + w + 64) * _NPAD + j * 1024,
                                         1024)])

    return k(m_p_flat, dst_s).reshape(HID, _NPAD)


def _global_mlp_kernel(aggr_ref, b2_ref, gw1_ref, gb1_ref, gw2_ref, gb2_ref,
                       o_ref):
    a = aggr_ref[...] + b2_ref[...]
    t = jnp.maximum(
        jax.lax.dot_general(gw1_ref[...], a, (((1,), (0,)), ((), ())),
                            preferred_element_type=jnp.float32)
        + gb1_ref[...], 0.0)
    o_ref[...] = jax.lax.dot_general(
        t, gw2_ref[...], (((0,), (1,)), ((), ())),
        preferred_element_type=jnp.float32) + gb2_ref[...]


def _global_mlp_t(aggr_t, lb2, gw1, gb1, gw2, gb2):
    blk = 1024
    grid = _NPAD // blk
    return pl.pallas_call(
        _global_mlp_kernel,
        grid=(grid,),
        in_specs=[
            pl.BlockSpec((HID, blk), lambda i: (0, i)),
            pl.BlockSpec((HID, 1), lambda i: (0, 0)),
            pl.BlockSpec((HID, HID), lambda i: (0, 0)),
            pl.BlockSpec((HID, 1), lambda i: (0, 0)),
            pl.BlockSpec((HID, HID), lambda i: (0, 0)),
            pl.BlockSpec((1, HID), lambda i: (0, 0)),
        ],
        out_specs=pl.BlockSpec((blk, HID), lambda i: (i, 0)),
        out_shape=jax.ShapeDtypeStruct((_NPAD, HID), jnp.float32),
    )(aggr_t, lb2.reshape(HID, 1), gw1, gb1.reshape(HID, 1), gw2,
      gb2.reshape(1, HID))


def _layer(h, pos, src2d, dstg2d, dst_s, lw1, lb1, lw2, lb2, gw1, gb1, gw2,
           gb2):
    a, b = _node_precompute(h, pos, lw1, lb1)
    z = _gather_z(a, b, src2d, dstg2d)
    m_p = _edge_matmul_packed(z, lw2)
    aggr_t = _scatter_max(m_p.reshape(-1), dst_s)
    return _global_mlp_t(aggr_t, lb2, gw1, gb1, gw2, gb2)[:N]


def kernel(x, pos, edge_index, l0_lw1, l0_lb1, l0_lw2, l0_lb2, l0_gw1,
           l0_gb1, l0_gw2, l0_gb2, l1_lw1, l1_lb1, l1_lw2, l1_lb2, l1_gw1,
           l1_gb1, l1_gw2, l1_gb2):
    n = x.shape[0]
    e = edge_index.shape[1]
    npad = _E2 - e - n
    nidx = _IDXROWS * _CH - e - n
    loop = jnp.arange(n, dtype=edge_index.dtype)
    src = jnp.concatenate(
        [edge_index[0], loop, jnp.zeros((nidx,), edge_index.dtype)])
    dst = jnp.concatenate([edge_index[1], loop])
    dst_g = jnp.concatenate([dst, jnp.zeros((nidx,), edge_index.dtype)])
    dst_s = jnp.concatenate([dst, jnp.full((npad,), n, edge_index.dtype)])
    src2d = src.reshape(_IDXROWS, _CH)
    dstg2d = dst_g.reshape(_IDXROWS, _CH)
    h = _layer(x, pos, src2d, dstg2d, dst_s, l0_lw1, l0_lb1, l0_lw2, l0_lb2,
               l0_gw1, l0_gb1, l0_gw2, l0_gb2)
    h = _layer(h, pos, src2d, dstg2d, dst_s, l1_lw1, l1_lb1, l1_lw2, l1_lb2,
               l1_gw1, l1_gb1, l1_gw2, l1_gb2)
    return h
